# Initial kernel scaffold; baseline (speedup 1.0000x reference)
import jax
import jax.numpy as jnp
from jax.experimental import pallas as pl


def kernel(x, edge_index, edge_type, basis0, comp0, root0, bias0, lnw0, lnb0, basis1, comp1, root1, bias1, lnw1, lnb1):
    raise NotImplementedError("write your pallas kernel here")


# speedup vs baseline: 9.9110x; 9.9110x over previous
import functools

import jax
import jax.numpy as jnp
from jax import lax
from jax.experimental import pallas as pl
from jax.experimental.pallas import tpu as pltpu
import jax.experimental.pallas.tpu_sc as plsc

N = 10000
E = 320000
D = 128
R = 3
HALF = D // 2
SEGS = R * N
SEGS_PAD = 30080
ROWS_PER_TILE = SEGS_PAD // 16

_CHA = 80
_CHB = 80


def _make_seg_counts():
    mesh = plsc.VectorSubcoreMesh(core_axis_name="c", subcore_axis_name="s")

    @functools.partial(
        pl.kernel,
        out_type=[
            jax.ShapeDtypeStruct((E,), jnp.int32),
            jax.ShapeDtypeStruct((2, SEGS_PAD, 16), jnp.float32),
        ],
        mesh=mesh,
        compiler_params=pltpu.CompilerParams(use_tc_tiling_on_sc=False),
        scratch_types=[
            pltpu.VMEM_SHARED((SEGS_PAD, 16), jnp.float32),
            pltpu.VMEM((_CHA,), jnp.int32),
            pltpu.VMEM((_CHA,), jnp.int32),
            pltpu.VMEM((_CHA,), jnp.int32),
            pltpu.VMEM((_CHA, 16), jnp.float32),
        ],
    )
    def k(dst_hbm, typ_hbm, ones_hbm, zc_hbm, seg_hbm, cnt_hbm,
          tab, dstv, typv, segv, onesv):
        cid = lax.axis_index("c")
        sid = lax.axis_index("s")
        wid = sid * 2 + cid
        pltpu.sync_copy(zc_hbm, tab.at[pl.ds(sid * ROWS_PER_TILE, ROWS_PER_TILE)])
        pltpu.sync_copy(ones_hbm, onesv)
        plsc.subcore_barrier()

        epw = E // 32
        nch = epw // _CHA

        def body(kk, _):
            base = wid * epw + kk * _CHA
            pltpu.sync_copy(dst_hbm.at[pl.ds(base, _CHA)], dstv)
            pltpu.sync_copy(typ_hbm.at[pl.ds(base, _CHA)], typv)
            for j in range(_CHA // 16):
                sl = pl.ds(j * 16, 16)
                segv[sl] = typv[sl] * N + dstv[sl]
            pltpu.sync_copy(segv, seg_hbm.at[pl.ds(base, _CHA)])
            pltpu.sync_copy(onesv, tab.at[segv], add=True)
            return _

        lax.fori_loop(0, nch, body, None)
        plsc.subcore_barrier()
        sl = pl.ds(sid * ROWS_PER_TILE, ROWS_PER_TILE)
        pltpu.sync_copy(tab.at[sl], cnt_hbm.at[cid, sl])

    return k


def _make_scatter():
    mesh = plsc.VectorSubcoreMesh(core_axis_name="c", subcore_axis_name="s")

    @functools.partial(
        pl.kernel,
        out_type=[jax.ShapeDtypeStruct((2, SEGS_PAD, HALF), jnp.float32)],
        mesh=mesh,
        compiler_params=pltpu.CompilerParams(use_tc_tiling_on_sc=False),
        scratch_types=[
            pltpu.VMEM_SHARED((SEGS_PAD, HALF), jnp.float32),
            pltpu.VMEM((_CHB,), jnp.int32),
            pltpu.VMEM((_CHB,), jnp.int32),
            pltpu.VMEM((_CHB, HALF), jnp.float32),
            pltpu.SemaphoreType.DMA,
        ],
    )
    def k(h0_hbm, h1_hbm, src_hbm, seg_hbm, ztab_hbm, s_hbm,
          tab, srcv, segv, rows, sem):
        cid = lax.axis_index("c")
        sid = lax.axis_index("s")
        pltpu.sync_copy(ztab_hbm, tab.at[pl.ds(sid * ROWS_PER_TILE, ROWS_PER_TILE)])
        plsc.subcore_barrier()

        epw = E // 16
        nch = epw // _CHB

        def run(h_hbm):
            def body(kk, _):
                base = sid * epw + kk * _CHB
                pltpu.sync_copy(src_hbm.at[pl.ds(base, _CHB)], srcv)
                pltpu.sync_copy(seg_hbm.at[pl.ds(base, _CHB)], segv)
                pltpu.async_copy(h_hbm.at[srcv], rows, sem).wait()
                pltpu.sync_copy(rows, tab.at[segv], add=True)
                return _
            lax.fori_loop(0, nch, body, None)

        @pl.when(cid == 0)
        def _():
            run(h0_hbm)

        @pl.when(cid == 1)
        def _():
            run(h1_hbm)

        plsc.subcore_barrier()
        sl = pl.ds(sid * ROWS_PER_TILE, ROWS_PER_TILE)
        pltpu.sync_copy(tab.at[sl], s_hbm.at[cid, sl])

    return k


_seg_counts_kernel = _make_seg_counts()
_scatter_kernel = _make_scatter()

_BN = 1000


def _dense_body(s_ref, cnt_ref, h_ref, basis_ref, comp_ref, root_ref,
                bias_ref, lnw_ref, lnb_ref, out_ref):
    h_b = h_ref[...]
    acc = jnp.dot(h_b, root_ref[...], preferred_element_type=jnp.float32)
    acc = acc + bias_ref[...]
    for r in range(R):
        w_r = comp_ref[r, 0] * basis_ref[0] + comp_ref[r, 1] * basis_ref[1]
        cnt = cnt_ref[r, :, 0:1] + cnt_ref[R + r, :, 0:1]
        inv = 1.0 / jnp.maximum(cnt, 1.0)
        m = jnp.dot(s_ref[r], w_r[:HALF, :], preferred_element_type=jnp.float32)
        m = m + jnp.dot(s_ref[R + r], w_r[HALF:, :],
                        preferred_element_type=jnp.float32)
        acc = acc + m * inv
    mu = jnp.mean(acc, axis=-1, keepdims=True)
    dlt = acc - mu
    var = jnp.mean(dlt * dlt, axis=-1, keepdims=True)
    y = dlt * lax.rsqrt(var + 1e-5) * lnw_ref[...] + lnb_ref[...]
    e = jnp.where(y > 0, y, jnp.exp(y) - 1.0)
    out_ref[...] = e + h_b


def _dense(s6, cnt6, h, basis, comp_pad, root, bias, lnw, lnb):
    grid = (N // _BN,)
    return pl.pallas_call(
        _dense_body,
        grid=grid,
        in_specs=[
            pl.BlockSpec((2 * R, _BN, HALF), lambda i: (0, i, 0)),
            pl.BlockSpec((2 * R, _BN, 16), lambda i: (0, i, 0)),
            pl.BlockSpec((_BN, D), lambda i: (i, 0)),
            pl.BlockSpec((2, D, D), lambda i: (0, 0, 0)),
            pl.BlockSpec((8, 128), lambda i: (0, 0)),
            pl.BlockSpec((D, D), lambda i: (0, 0)),
            pl.BlockSpec((1, D), lambda i: (0, 0)),
            pl.BlockSpec((1, D), lambda i: (0, 0)),
            pl.BlockSpec((1, D), lambda i: (0, 0)),
        ],
        out_specs=pl.BlockSpec((_BN, D), lambda i: (i, 0)),
        out_shape=jax.ShapeDtypeStruct((N, D), jnp.float32),
    )(s6, cnt6, h, basis, comp_pad, root, bias, lnw, lnb)


def kernel(x, edge_index, edge_type, basis0, comp0, root0, bias0, lnw0, lnb0,
           basis1, comp1, root1, bias1, lnw1, lnb1):
    src = edge_index[0].astype(jnp.int32)
    dst = edge_index[1].astype(jnp.int32)
    typ = edge_type.astype(jnp.int32)

    ones = jnp.ones((_CHA, 16), jnp.float32)
    zc = jnp.zeros((ROWS_PER_TILE, 16), jnp.float32)
    ztab = jnp.zeros((ROWS_PER_TILE, HALF), jnp.float32)

    seg, counts = _seg_counts_kernel(dst, typ, ones, zc)
    cnt6 = counts[:, :SEGS].reshape(2 * R, N, 16)

    h = x
    for basis, comp, root, bias, lnw, lnb in (
        (basis0, comp0, root0, bias0, lnw0, lnb0),
        (basis1, comp1, root1, bias1, lnw1, lnb1),
    ):
        h0 = h[:, :HALF]
        h1 = h[:, HALF:]
        (s,) = _scatter_kernel(h0, h1, src, seg, ztab)
        s6 = s[:, :SEGS].reshape(2 * R, N, HALF)
        comp_pad = jnp.zeros((8, 128), jnp.float32).at[:R, :2].set(comp)
        h = _dense(s6, cnt6, h, basis, comp_pad, root,
                   bias[None, :], lnw[None, :], lnb[None, :])
    return h


# speedup vs baseline: 13.9281x; 1.4053x over previous
import functools

import jax
import jax.numpy as jnp
from jax import lax
from jax.experimental import pallas as pl
from jax.experimental.pallas import tpu as pltpu
import jax.experimental.pallas.tpu_sc as plsc

N = 10000
E = 320000
D = 128
R = 3
HALF = D // 2
SEGS = R * N
SEGS_PAD = 30080
ROWS_PER_TILE = SEGS_PAD // 16

_CHA = 80
_CHB = 80
_NCHB = (E // 16) // _CHB


def _make_seg_counts():
    mesh = plsc.VectorSubcoreMesh(core_axis_name="c", subcore_axis_name="s")

    @functools.partial(
        pl.kernel,
        out_type=[
            jax.ShapeDtypeStruct((E,), jnp.int32),
            jax.ShapeDtypeStruct((2, SEGS_PAD, 16), jnp.float32),
        ],
        mesh=mesh,
        compiler_params=pltpu.CompilerParams(use_tc_tiling_on_sc=False),
        scratch_types=[
            pltpu.VMEM_SHARED((SEGS_PAD, 16), jnp.float32),
            pltpu.VMEM((_CHA,), jnp.int32),
            pltpu.VMEM((_CHA,), jnp.int32),
            pltpu.VMEM((_CHA,), jnp.int32),
            pltpu.VMEM((_CHA, 16), jnp.float32),
        ],
    )
    def k(dst_hbm, typ_hbm, ones_hbm, zc_hbm, seg_hbm, cnt_hbm,
          tab, dstv, typv, segv, onesv):
        cid = lax.axis_index("c")
        sid = lax.axis_index("s")
        wid = sid * 2 + cid
        pltpu.sync_copy(zc_hbm, tab.at[pl.ds(sid * ROWS_PER_TILE, ROWS_PER_TILE)])
        pltpu.sync_copy(ones_hbm, onesv)
        plsc.subcore_barrier()

        epw = E // 32
        nch = epw // _CHA

        def body(kk, _):
            base = wid * epw + kk * _CHA
            pltpu.sync_copy(dst_hbm.at[pl.ds(base, _CHA)], dstv)
            pltpu.sync_copy(typ_hbm.at[pl.ds(base, _CHA)], typv)
            for j in range(_CHA // 16):
                sl = pl.ds(j * 16, 16)
                segv[sl] = typv[sl] * N + dstv[sl]
            pltpu.sync_copy(segv, seg_hbm.at[pl.ds(base, _CHA)])
            pltpu.sync_copy(onesv, tab.at[segv], add=True)
            return _

        lax.fori_loop(0, nch, body, None)
        plsc.subcore_barrier()
        sl = pl.ds(sid * ROWS_PER_TILE, ROWS_PER_TILE)
        pltpu.sync_copy(tab.at[sl], cnt_hbm.at[cid, sl])

    return k


def _make_scatter():
    mesh = plsc.VectorSubcoreMesh(core_axis_name="c", subcore_axis_name="s")

    @functools.partial(
        pl.kernel,
        out_type=[jax.ShapeDtypeStruct((2, SEGS_PAD, HALF), jnp.float32)],
        mesh=mesh,
        compiler_params=pltpu.CompilerParams(use_tc_tiling_on_sc=False),
        scratch_types=[
            pltpu.VMEM_SHARED((SEGS_PAD, HALF), jnp.float32),
            pltpu.VMEM((_CHB,), jnp.int32),
            pltpu.VMEM((_CHB,), jnp.int32),
            pltpu.VMEM((_CHB,), jnp.int32),
            pltpu.VMEM((_CHB,), jnp.int32),
            pltpu.VMEM((_CHB, HALF), jnp.float32),
            pltpu.VMEM((_CHB, HALF), jnp.float32),
            pltpu.SemaphoreType.DMA,
            pltpu.SemaphoreType.DMA,
        ],
    )
    def k(h0_hbm, h1_hbm, src2_hbm, seg2_hbm, ztab_hbm, s_hbm,
          tab, srcv0, srcv1, segv0, segv1, rows0, rows1, semg0, semg1):
        cid = lax.axis_index("c")
        sid = lax.axis_index("s")
        pltpu.sync_copy(ztab_hbm, tab.at[pl.ds(sid * ROWS_PER_TILE, ROWS_PER_TILE)])
        plsc.subcore_barrier()
        c0 = sid * _NCHB

        def run(h_hbm):
            pltpu.sync_copy(src2_hbm.at[c0], srcv0)
            pltpu.sync_copy(seg2_hbm.at[c0], segv0)
            pltpu.async_copy(h_hbm.at[srcv0], rows0, semg0)

            def body(g, _):
                k1 = c0 + 2 * g + 1
                pltpu.sync_copy(src2_hbm.at[k1], srcv1)
                pltpu.sync_copy(seg2_hbm.at[k1], segv1)
                pltpu.async_copy(h_hbm.at[srcv1], rows1, semg1)

                pltpu.make_async_copy(h_hbm.at[srcv0], rows0, semg0).wait()
                pltpu.sync_copy(rows0, tab.at[segv0], add=True)

                @pl.when(g + 1 < _NCHB // 2)
                def _():
                    pltpu.sync_copy(src2_hbm.at[k1 + 1], srcv0)
                    pltpu.sync_copy(seg2_hbm.at[k1 + 1], segv0)
                    pltpu.async_copy(h_hbm.at[srcv0], rows0, semg0)

                pltpu.make_async_copy(h_hbm.at[srcv1], rows1, semg1).wait()
                pltpu.sync_copy(rows1, tab.at[segv1], add=True)
                return _

            lax.fori_loop(0, _NCHB // 2, body, None)

        @pl.when(cid == 0)
        def _():
            run(h0_hbm)

        @pl.when(cid == 1)
        def _():
            run(h1_hbm)

        plsc.subcore_barrier()
        sl = pl.ds(sid * ROWS_PER_TILE, ROWS_PER_TILE)
        pltpu.sync_copy(tab.at[sl], s_hbm.at[cid, sl])

    return k


_seg_counts_kernel = _make_seg_counts()
_scatter_kernel = _make_scatter()

_BN = 1000


def _dense_body(s_ref, cnt_ref, h_ref, basis_ref, comp_ref, root_ref,
                bias_ref, lnw_ref, lnb_ref, out_ref):
    h_b = h_ref[...]
    acc = jnp.dot(h_b, root_ref[...], preferred_element_type=jnp.float32)
    acc = acc + bias_ref[...]
    for r in range(R):
        w_r = comp_ref[r, 0] * basis_ref[0] + comp_ref[r, 1] * basis_ref[1]
        cnt = cnt_ref[r, :, 0:1] + cnt_ref[R + r, :, 0:1]
        inv = 1.0 / jnp.maximum(cnt, 1.0)
        m = jnp.dot(s_ref[r], w_r[:HALF, :], preferred_element_type=jnp.float32)
        m = m + jnp.dot(s_ref[R + r], w_r[HALF:, :],
                        preferred_element_type=jnp.float32)
        acc = acc + m * inv
    mu = jnp.mean(acc, axis=-1, keepdims=True)
    dlt = acc - mu
    var = jnp.mean(dlt * dlt, axis=-1, keepdims=True)
    y = dlt * lax.rsqrt(var + 1e-5) * lnw_ref[...] + lnb_ref[...]
    e = jnp.where(y > 0, y, jnp.exp(y) - 1.0)
    out_ref[...] = e + h_b


def _dense(s6, cnt6, h, basis, comp_pad, root, bias, lnw, lnb):
    grid = (N // _BN,)
    return pl.pallas_call(
        _dense_body,
        grid=grid,
        in_specs=[
            pl.BlockSpec((2 * R, _BN, HALF), lambda i: (0, i, 0)),
            pl.BlockSpec((2 * R, _BN, 16), lambda i: (0, i, 0)),
            pl.BlockSpec((_BN, D), lambda i: (i, 0)),
            pl.BlockSpec((2, D, D), lambda i: (0, 0, 0)),
            pl.BlockSpec((8, 128), lambda i: (0, 0)),
            pl.BlockSpec((D, D), lambda i: (0, 0)),
            pl.BlockSpec((1, D), lambda i: (0, 0)),
            pl.BlockSpec((1, D), lambda i: (0, 0)),
            pl.BlockSpec((1, D), lambda i: (0, 0)),
        ],
        out_specs=pl.BlockSpec((_BN, D), lambda i: (i, 0)),
        out_shape=jax.ShapeDtypeStruct((N, D), jnp.float32),
    )(s6, cnt6, h, basis, comp_pad, root, bias, lnw, lnb)


def kernel(x, edge_index, edge_type, basis0, comp0, root0, bias0, lnw0, lnb0,
           basis1, comp1, root1, bias1, lnw1, lnb1):
    src = edge_index[0].astype(jnp.int32)
    dst = edge_index[1].astype(jnp.int32)
    typ = edge_type.astype(jnp.int32)

    ones = jnp.ones((_CHA, 16), jnp.float32)
    zc = jnp.zeros((ROWS_PER_TILE, 16), jnp.float32)
    ztab = jnp.zeros((ROWS_PER_TILE, HALF), jnp.float32)

    seg, counts = _seg_counts_kernel(dst, typ, ones, zc)
    cnt6 = counts[:, :SEGS].reshape(2 * R, N, 16)
    src2 = src.reshape(E // _CHB, _CHB)
    seg2 = seg.reshape(E // _CHB, _CHB)

    h = x
    for basis, comp, root, bias, lnw, lnb in (
        (basis0, comp0, root0, bias0, lnw0, lnb0),
        (basis1, comp1, root1, bias1, lnw1, lnb1),
    ):
        h0 = h[:, :HALF]
        h1 = h[:, HALF:]
        (s,) = _scatter_kernel(h0, h1, src2, seg2, ztab)
        s6 = s[:, :SEGS].reshape(2 * R, N, HALF)
        comp_pad = jnp.zeros((8, 128), jnp.float32).at[:R, :2].set(comp)
        h = _dense(s6, cnt6, h, basis, comp_pad, root,
                   bias[None, :], lnw[None, :], lnb[None, :])
    return h


# speedup vs baseline: 18.6123x; 1.3363x over previous
import functools

import jax
import jax.numpy as jnp
from jax import lax
from jax.experimental import pallas as pl
from jax.experimental.pallas import tpu as pltpu
import jax.experimental.pallas.tpu_sc as plsc

N = 10000
E = 320000
D = 128
R = 3
HALF = D // 2
SEGS = R * N
SEGS_PAD = 30016
ROWS_PER_TILE = SEGS_PAD // 16

_CHA = 80
_CHB = 80
_NCHB = (E // 16) // _CHB


def _make_seg_counts():
    mesh = plsc.VectorSubcoreMesh(core_axis_name="c", subcore_axis_name="s")

    @functools.partial(
        pl.kernel,
        out_type=[
            jax.ShapeDtypeStruct((E,), jnp.int32),
            jax.ShapeDtypeStruct((2, SEGS_PAD, 16), jnp.float32),
        ],
        mesh=mesh,
        compiler_params=pltpu.CompilerParams(use_tc_tiling_on_sc=False),
        scratch_types=[
            pltpu.VMEM_SHARED((SEGS_PAD, 16), jnp.float32),
            pltpu.VMEM((_CHA,), jnp.int32),
            pltpu.VMEM((_CHA,), jnp.int32),
            pltpu.VMEM((_CHA,), jnp.int32),
            pltpu.VMEM((_CHA, 16), jnp.float32),
        ],
    )
    def k(dst_hbm, typ_hbm, ones_hbm, zc_hbm, seg_hbm, cnt_hbm,
          tab, dstv, typv, segv, onesv):
        cid = lax.axis_index("c")
        sid = lax.axis_index("s")
        wid = sid * 2 + cid
        pltpu.sync_copy(zc_hbm, tab.at[pl.ds(sid * ROWS_PER_TILE, ROWS_PER_TILE)])
        pltpu.sync_copy(ones_hbm, onesv)
        plsc.subcore_barrier()

        epw = E // 32
        nch = epw // _CHA

        def body(kk, _):
            base = wid * epw + kk * _CHA
            pltpu.sync_copy(dst_hbm.at[pl.ds(base, _CHA)], dstv)
            pltpu.sync_copy(typ_hbm.at[pl.ds(base, _CHA)], typv)
            for j in range(_CHA // 16):
                sl = pl.ds(j * 16, 16)
                segv[sl] = typv[sl] * N + dstv[sl]
            pltpu.sync_copy(segv, seg_hbm.at[pl.ds(base, _CHA)])
            pltpu.sync_copy(onesv, tab.at[segv], add=True)
            return _

        lax.fori_loop(0, nch, body, None)
        plsc.subcore_barrier()
        sl = pl.ds(sid * ROWS_PER_TILE, ROWS_PER_TILE)
        pltpu.sync_copy(tab.at[sl], cnt_hbm.at[cid, sl])

    return k


def _make_scatter():
    mesh = plsc.VectorSubcoreMesh(core_axis_name="c", subcore_axis_name="s")

    @functools.partial(
        pl.kernel,
        out_type=[jax.ShapeDtypeStruct((2, SEGS_PAD, HALF), jnp.float32)],
        mesh=mesh,
        compiler_params=pltpu.CompilerParams(use_tc_tiling_on_sc=False),
        scratch_types=[
            pltpu.VMEM_SHARED((SEGS_PAD, HALF), jnp.float32),
            pltpu.VMEM((_CHB,), jnp.int32),
            pltpu.VMEM((_CHB,), jnp.int32),
            pltpu.VMEM((_CHB,), jnp.int32),
            pltpu.VMEM((_CHB,), jnp.int32),
            pltpu.VMEM((_CHB,), jnp.int32),
            pltpu.VMEM((_CHB,), jnp.int32),
            pltpu.VMEM((_CHB,), jnp.int32),
            pltpu.VMEM((_CHB,), jnp.int32),
            pltpu.VMEM((_CHB, HALF), jnp.float32),
            pltpu.VMEM((_CHB, HALF), jnp.float32),
            pltpu.SemaphoreType.DMA,
            pltpu.SemaphoreType.DMA,
            pltpu.SemaphoreType.DMA,
            pltpu.SemaphoreType.DMA,
            pltpu.SemaphoreType.DMA,
            pltpu.SemaphoreType.DMA,
        ],
    )
    def k(h0_hbm, h1_hbm, src2_hbm, seg2_hbm, ztab_hbm, s_hbm,
          tab, srcva, srcvb, srcvc, srcvd, segva, segvb, segvc, segvd,
          rows0, rows1, semi0, semi1, semi2, semi3, semg0, semg1):
        cid = lax.axis_index("c")
        sid = lax.axis_index("s")
        pltpu.sync_copy(ztab_hbm, tab.at[pl.ds(sid * ROWS_PER_TILE, ROWS_PER_TILE)])
        plsc.subcore_barrier()
        c0 = sid * _NCHB
        srcv = (srcva, srcvb, srcvc, srcvd)
        segv = (segva, segvb, segvc, segvd)
        semi = (semi0, semi1, semi2, semi3)

        def prefetch(j, q):
            pltpu.async_copy(src2_hbm.at[q], srcv[j], semi[j])
            pltpu.async_copy(seg2_hbm.at[q], segv[j], semi[j])

        def wait_idx(j):
            pltpu.make_async_copy(src2_hbm.at[c0], srcv[j], semi[j]).wait()
            pltpu.make_async_copy(seg2_hbm.at[c0], segv[j], semi[j]).wait()

        def run(h_hbm):
            for j in range(4):
                prefetch(j, c0 + j)

            def gather(j, rows_p, semg_p):
                wait_idx(j)
                pltpu.async_copy(h_hbm.at[srcv[j]], rows_p, semg_p)

            def scat(j, rows_p, semg_p):
                pltpu.make_async_copy(
                    h_hbm.at[srcv[j]], rows_p, semg_p).wait()
                pltpu.sync_copy(rows_p, tab.at[segv[j]], add=True)

            def body(g, _):
                c = c0 + g * 4
                gather(0, rows0, semg0)
                gather(1, rows1, semg1)
                scat(0, rows0, semg0)
                prefetch(0, c + 4)
                gather(2, rows0, semg0)
                scat(1, rows1, semg1)
                prefetch(1, c + 5)
                gather(3, rows1, semg1)
                scat(2, rows0, semg0)

                @pl.when(g + 1 < _NCHB // 4)
                def _():
                    prefetch(2, c + 6)

                scat(3, rows1, semg1)

                @pl.when(g + 1 < _NCHB // 4)
                def _():
                    prefetch(3, c + 7)
                return _

            lax.fori_loop(0, _NCHB // 4, body, None)

            gather(0, rows0, semg0)
            gather(1, rows1, semg1)
            scat(0, rows0, semg0)
            scat(1, rows1, semg1)

        @pl.when(cid == 0)
        def _():
            run(h0_hbm)

        @pl.when(cid == 1)
        def _():
            run(h1_hbm)

        plsc.subcore_barrier()
        sl = pl.ds(sid * ROWS_PER_TILE, ROWS_PER_TILE)
        pltpu.sync_copy(tab.at[sl], s_hbm.at[cid, sl])

    return k


_seg_counts_kernel = _make_seg_counts()
_scatter_kernel = _make_scatter()

_BN = 1000


def _dense_body(s_ref, cnt_ref, h_ref, basis_ref, comp_ref, root_ref,
                bias_ref, lnw_ref, lnb_ref, out_ref):
    h_b = h_ref[...]
    acc = jnp.dot(h_b, root_ref[...], preferred_element_type=jnp.float32)
    acc = acc + bias_ref[...]
    for r in range(R):
        w_r = comp_ref[r, 0] * basis_ref[0] + comp_ref[r, 1] * basis_ref[1]
        cnt = cnt_ref[r, :, 0:1] + cnt_ref[R + r, :, 0:1]
        inv = 1.0 / jnp.maximum(cnt, 1.0)
        m = jnp.dot(s_ref[r], w_r[:HALF, :], preferred_element_type=jnp.float32)
        m = m + jnp.dot(s_ref[R + r], w_r[HALF:, :],
                        preferred_element_type=jnp.float32)
        acc = acc + m * inv
    mu = jnp.mean(acc, axis=-1, keepdims=True)
    dlt = acc - mu
    var = jnp.mean(dlt * dlt, axis=-1, keepdims=True)
    y = dlt * lax.rsqrt(var + 1e-5) * lnw_ref[...] + lnb_ref[...]
    e = jnp.where(y > 0, y, jnp.exp(y) - 1.0)
    out_ref[...] = e + h_b


def _dense(s6, cnt6, h, basis, comp_pad, root, bias, lnw, lnb):
    grid = (N // _BN,)
    return pl.pallas_call(
        _dense_body,
        grid=grid,
        in_specs=[
            pl.BlockSpec((2 * R, _BN, HALF), lambda i: (0, i, 0)),
            pl.BlockSpec((2 * R, _BN, 16), lambda i: (0, i, 0)),
            pl.BlockSpec((_BN, D), lambda i: (i, 0)),
            pl.BlockSpec((2, D, D), lambda i: (0, 0, 0)),
            pl.BlockSpec((8, 128), lambda i: (0, 0)),
            pl.BlockSpec((D, D), lambda i: (0, 0)),
            pl.BlockSpec((1, D), lambda i: (0, 0)),
            pl.BlockSpec((1, D), lambda i: (0, 0)),
            pl.BlockSpec((1, D), lambda i: (0, 0)),
        ],
        out_specs=pl.BlockSpec((_BN, D), lambda i: (i, 0)),
        out_shape=jax.ShapeDtypeStruct((N, D), jnp.float32),
    )(s6, cnt6, h, basis, comp_pad, root, bias, lnw, lnb)


def kernel(x, edge_index, edge_type, basis0, comp0, root0, bias0, lnw0, lnb0,
           basis1, comp1, root1, bias1, lnw1, lnb1):
    src = edge_index[0].astype(jnp.int32)
    dst = edge_index[1].astype(jnp.int32)
    typ = edge_type.astype(jnp.int32)

    ones = jnp.ones((_CHA, 16), jnp.float32)
    zc = jnp.zeros((ROWS_PER_TILE, 16), jnp.float32)
    ztab = jnp.zeros((ROWS_PER_TILE, HALF), jnp.float32)

    seg, counts = _seg_counts_kernel(dst, typ, ones, zc)
    cnt6 = counts[:, :SEGS].reshape(2 * R, N, 16)
    src2 = src.reshape(E // _CHB, _CHB)
    seg2 = seg.reshape(E // _CHB, _CHB)

    h = x
    for basis, comp, root, bias, lnw, lnb in (
        (basis0, comp0, root0, bias0, lnw0, lnb0),
        (basis1, comp1, root1, bias1, lnw1, lnb1),
    ):
        h0 = h[:, :HALF]
        h1 = h[:, HALF:]
        (s,) = _scatter_kernel(h0, h1, src2, seg2, ztab)
        s6 = s[:, :SEGS].reshape(2 * R, N, HALF)
        comp_pad = jnp.zeros((8, 128), jnp.float32).at[:R, :2].set(comp)
        h = _dense(s6, cnt6, h, basis, comp_pad, root,
                   bias[None, :], lnw[None, :], lnb[None, :])
    return h


# speedup vs baseline: 22.2489x; 1.1954x over previous
import functools

import jax
import jax.numpy as jnp
from jax import lax
from jax.experimental import pallas as pl
from jax.experimental.pallas import tpu as pltpu
import jax.experimental.pallas.tpu_sc as plsc

N = 10000
E = 320000
D = 128
R = 3
HALF = D // 2
SEGS = R * N
SEGS_PAD = 30016
ROWS_PER_TILE = SEGS_PAD // 16

_CHA = 400
_NCHA = (E // 32) // _CHA
_CHB = 80
_NCHB = (E // 16) // _CHB


def _make_seg_counts():
    mesh = plsc.VectorSubcoreMesh(core_axis_name="c", subcore_axis_name="s")

    @functools.partial(
        pl.kernel,
        out_type=[
            jax.ShapeDtypeStruct((E // _CHA, _CHA), jnp.int32),
            jax.ShapeDtypeStruct((2, SEGS_PAD, 16), jnp.float32),
        ],
        mesh=mesh,
        compiler_params=pltpu.CompilerParams(use_tc_tiling_on_sc=False),
        scratch_types=[
            pltpu.VMEM_SHARED((SEGS_PAD, 16), jnp.float32),
            pltpu.VMEM((_CHA,), jnp.int32),
            pltpu.VMEM((_CHA,), jnp.int32),
            pltpu.VMEM((_CHA,), jnp.int32),
            pltpu.VMEM((_CHA,), jnp.int32),
            pltpu.VMEM((_CHA,), jnp.int32),
            pltpu.VMEM((_CHA,), jnp.int32),
            pltpu.VMEM((_CHA, 16), jnp.float32),
            pltpu.SemaphoreType.DMA,
            pltpu.SemaphoreType.DMA,
            pltpu.SemaphoreType.DMA,
            pltpu.SemaphoreType.DMA,
        ],
    )
    def k(dst2_hbm, typ2_hbm, ones_hbm, zc_hbm, seg2_hbm, cnt_hbm,
          tab, dstv0, dstv1, typv0, typv1, segv0, segv1, onesv,
          semi0, semi1, semw0, semw1):
        cid = lax.axis_index("c")
        sid = lax.axis_index("s")
        wid = sid * 2 + cid
        c0 = wid * _NCHA
        dstv = (dstv0, dstv1)
        typv = (typv0, typv1)
        segv = (segv0, segv1)
        semi = (semi0, semi1)
        semw = (semw0, semw1)

        def prefetch(b, q):
            pltpu.async_copy(dst2_hbm.at[q], dstv[b], semi[b])
            pltpu.async_copy(typ2_hbm.at[q], typv[b], semi[b])

        def wait_idx(b):
            pltpu.make_async_copy(dst2_hbm.at[c0], dstv[b], semi[b]).wait()
            pltpu.make_async_copy(typ2_hbm.at[c0], typv[b], semi[b]).wait()

        def compute_seg(b):
            for j in range(_CHA // 16):
                sl = pl.ds(j * 16, 16)
                segv[b][sl] = typv[b][sl] * N + dstv[b][sl]

        def writeback(b, q):
            pltpu.async_copy(segv[b], seg2_hbm.at[q], semw[b])

        def wait_wb(b):
            pltpu.make_async_copy(seg2_hbm.at[c0], segv[b], semw[b]).wait()

        def scat_ones(b):
            pltpu.sync_copy(onesv, tab.at[segv[b]], add=True)

        prefetch(0, c0)
        prefetch(1, c0 + 1)
        pltpu.sync_copy(zc_hbm, tab.at[pl.ds(sid * ROWS_PER_TILE, ROWS_PER_TILE)])
        pltpu.sync_copy(ones_hbm, onesv)
        plsc.subcore_barrier()

        for b in range(2):
            wait_idx(b)
            compute_seg(b)
            prefetch(b, c0 + 2 + b)
            writeback(b, c0 + b)
            scat_ones(b)

        def body(g, _):
            for b in range(2):
                kk = 2 * g + b
                wait_idx(b)
                wait_wb(b)
                compute_seg(b)

                @pl.when(kk + 2 < _NCHA)
                def _():
                    prefetch(b, c0 + kk + 2)

                writeback(b, c0 + kk)
                scat_ones(b)
            return _

        lax.fori_loop(1, _NCHA // 2, body, None)

        wait_idx(0)
        wait_wb(0)
        compute_seg(0)
        writeback(0, c0 + _NCHA - 1)
        scat_ones(0)

        wait_wb(0)
        wait_wb(1)
        plsc.subcore_barrier()
        sl = pl.ds(sid * ROWS_PER_TILE, ROWS_PER_TILE)
        pltpu.sync_copy(tab.at[sl], cnt_hbm.at[cid, sl])

    return k


def _make_scatter():
    mesh = plsc.VectorSubcoreMesh(core_axis_name="c", subcore_axis_name="s")

    @functools.partial(
        pl.kernel,
        out_type=[jax.ShapeDtypeStruct((2, SEGS_PAD, HALF), jnp.float32)],
        mesh=mesh,
        compiler_params=pltpu.CompilerParams(use_tc_tiling_on_sc=False),
        scratch_types=[
            pltpu.VMEM_SHARED((SEGS_PAD, HALF), jnp.float32),
            pltpu.VMEM((_CHB,), jnp.int32),
            pltpu.VMEM((_CHB,), jnp.int32),
            pltpu.VMEM((_CHB,), jnp.int32),
            pltpu.VMEM((_CHB,), jnp.int32),
            pltpu.VMEM((_CHB,), jnp.int32),
            pltpu.VMEM((_CHB,), jnp.int32),
            pltpu.VMEM((_CHB,), jnp.int32),
            pltpu.VMEM((_CHB,), jnp.int32),
            pltpu.VMEM((_CHB, HALF), jnp.float32),
            pltpu.VMEM((_CHB, HALF), jnp.float32),
            pltpu.SemaphoreType.DMA,
            pltpu.SemaphoreType.DMA,
            pltpu.SemaphoreType.DMA,
            pltpu.SemaphoreType.DMA,
            pltpu.SemaphoreType.DMA,
            pltpu.SemaphoreType.DMA,
        ],
    )
    def k(h0_hbm, h1_hbm, src2_hbm, seg2_hbm, ztab_hbm, s_hbm,
          tab, srcva, srcvb, srcvc, srcvd, segva, segvb, segvc, segvd,
          rows0, rows1, semi0, semi1, semi2, semi3, semg0, semg1):
        cid = lax.axis_index("c")
        sid = lax.axis_index("s")
        pltpu.sync_copy(ztab_hbm, tab.at[pl.ds(sid * ROWS_PER_TILE, ROWS_PER_TILE)])
        plsc.subcore_barrier()
        c0 = sid * _NCHB
        srcv = (srcva, srcvb, srcvc, srcvd)
        segv = (segva, segvb, segvc, segvd)
        semi = (semi0, semi1, semi2, semi3)

        def prefetch(j, q):
            pltpu.async_copy(src2_hbm.at[q], srcv[j], semi[j])
            pltpu.async_copy(seg2_hbm.at[q], segv[j], semi[j])

        def wait_idx(j):
            pltpu.make_async_copy(src2_hbm.at[c0], srcv[j], semi[j]).wait()
            pltpu.make_async_copy(seg2_hbm.at[c0], segv[j], semi[j]).wait()

        def run(h_hbm):
            for j in range(4):
                prefetch(j, c0 + j)

            def gather(j, rows_p, semg_p):
                wait_idx(j)
                pltpu.async_copy(h_hbm.at[srcv[j]], rows_p, semg_p)

            def scat(j, rows_p, semg_p):
                pltpu.make_async_copy(
                    h_hbm.at[srcv[j]], rows_p, semg_p).wait()
                pltpu.sync_copy(rows_p, tab.at[segv[j]], add=True)

            def body(g, _):
                c = c0 + g * 4
                gather(0, rows0, semg0)
                gather(1, rows1, semg1)
                scat(0, rows0, semg0)
                prefetch(0, c + 4)
                gather(2, rows0, semg0)
                scat(1, rows1, semg1)
                prefetch(1, c + 5)
                gather(3, rows1, semg1)
                scat(2, rows0, semg0)

                @pl.when(g + 1 < _NCHB // 4)
                def _():
                    prefetch(2, c + 6)

                scat(3, rows1, semg1)

                @pl.when(g + 1 < _NCHB // 4)
                def _():
                    prefetch(3, c + 7)
                return _

            lax.fori_loop(0, _NCHB // 4, body, None)

            gather(0, rows0, semg0)
            gather(1, rows1, semg1)
            scat(0, rows0, semg0)
            scat(1, rows1, semg1)

        @pl.when(cid == 0)
        def _():
            run(h0_hbm)

        @pl.when(cid == 1)
        def _():
            run(h1_hbm)

        plsc.subcore_barrier()
        sl = pl.ds(sid * ROWS_PER_TILE, ROWS_PER_TILE)
        pltpu.sync_copy(tab.at[sl], s_hbm.at[cid, sl])

    return k


_seg_counts_kernel = _make_seg_counts()
_scatter_kernel = _make_scatter()

_BN = 1000


def _dense_body(s_ref, cnt_ref, h_ref, basis_ref, comp_ref, root_ref,
                bias_ref, lnw_ref, lnb_ref, out_ref):
    h_b = h_ref[...]
    acc = jnp.dot(h_b, root_ref[...], preferred_element_type=jnp.float32)
    acc = acc + bias_ref[...]
    for r in range(R):
        w_r = comp_ref[r, 0] * basis_ref[0] + comp_ref[r, 1] * basis_ref[1]
        cnt = cnt_ref[r, :, 0:1] + cnt_ref[R + r, :, 0:1]
        inv = 1.0 / jnp.maximum(cnt, 1.0)
        m = jnp.dot(s_ref[r], w_r[:HALF, :], preferred_element_type=jnp.float32)
        m = m + jnp.dot(s_ref[R + r], w_r[HALF:, :],
                        preferred_element_type=jnp.float32)
        acc = acc + m * inv
    mu = jnp.mean(acc, axis=-1, keepdims=True)
    dlt = acc - mu
    var = jnp.mean(dlt * dlt, axis=-1, keepdims=True)
    y = dlt * lax.rsqrt(var + 1e-5) * lnw_ref[...] + lnb_ref[...]
    e = jnp.where(y > 0, y, jnp.exp(y) - 1.0)
    out_ref[...] = e + h_b


def _dense(s6, cnt6, h, basis, comp_pad, root, bias, lnw, lnb):
    grid = (N // _BN,)
    return pl.pallas_call(
        _dense_body,
        grid=grid,
        in_specs=[
            pl.BlockSpec((2 * R, _BN, HALF), lambda i: (0, i, 0)),
            pl.BlockSpec((2 * R, _BN, 16), lambda i: (0, i, 0)),
            pl.BlockSpec((_BN, D), lambda i: (i, 0)),
            pl.BlockSpec((2, D, D), lambda i: (0, 0, 0)),
            pl.BlockSpec((8, 128), lambda i: (0, 0)),
            pl.BlockSpec((D, D), lambda i: (0, 0)),
            pl.BlockSpec((1, D), lambda i: (0, 0)),
            pl.BlockSpec((1, D), lambda i: (0, 0)),
            pl.BlockSpec((1, D), lambda i: (0, 0)),
        ],
        out_specs=pl.BlockSpec((_BN, D), lambda i: (i, 0)),
        out_shape=jax.ShapeDtypeStruct((N, D), jnp.float32),
    )(s6, cnt6, h, basis, comp_pad, root, bias, lnw, lnb)


def kernel(x, edge_index, edge_type, basis0, comp0, root0, bias0, lnw0, lnb0,
           basis1, comp1, root1, bias1, lnw1, lnb1):
    src = edge_index[0].astype(jnp.int32)
    dst = edge_index[1].astype(jnp.int32)
    typ = edge_type.astype(jnp.int32)

    ones = jnp.ones((_CHA, 16), jnp.float32)
    zc = jnp.zeros((ROWS_PER_TILE, 16), jnp.float32)
    ztab = jnp.zeros((ROWS_PER_TILE, HALF), jnp.float32)

    sega, counts = _seg_counts_kernel(
        dst.reshape(E // _CHA, _CHA), typ.reshape(E // _CHA, _CHA), ones, zc)
    cnt6 = counts[:, :SEGS].reshape(2 * R, N, 16)
    src2 = src.reshape(E // _CHB, _CHB)
    seg2 = sega.reshape(E // _CHB, _CHB)

    h = x
    for basis, comp, root, bias, lnw, lnb in (
        (basis0, comp0, root0, bias0, lnw0, lnb0),
        (basis1, comp1, root1, bias1, lnw1, lnb1),
    ):
        h0 = h[:, :HALF]
        h1 = h[:, HALF:]
        (s,) = _scatter_kernel(h0, h1, src2, seg2, ztab)
        s6 = s[:, :SEGS].reshape(2 * R, N, HALF)
        comp_pad = jnp.zeros((8, 128), jnp.float32).at[:R, :2].set(comp)
        h = _dense(s6, cnt6, h, basis, comp_pad, root,
                   bias[None, :], lnw[None, :], lnb[None, :])
    return h
